# baseline (speedup 1.0000x reference)
import functools

import jax
import jax.numpy as jnp
from jax import lax
from jax.experimental import pallas as pl
from jax.experimental.pallas import tpu as pltpu
from jax.experimental.pallas import tpu_sc as plsc

NC = 2
NS = 16
L = 16
NW = NC * NS

ROWS, COLS, D = 16384, 200, 3
W_COLS = ROWS // NW
RB = 8
NCH = COLS // RB
GRP = RB * W_COLS // L

_mesh = plsc.VectorSubcoreMesh(
    core_axis_name="c", subcore_axis_name="s", num_cores=NC, num_subcores=NS
)


@functools.partial(
    pl.kernel,
    out_type=jax.ShapeDtypeStruct((D, COLS, ROWS), jnp.float32),
    mesh=_mesh,
    compiler_params=pltpu.CompilerParams(
        needs_layout_passes=False, use_tc_tiling_on_sc=True
    ),
    scratch_types=[
        pltpu.VMEM((RB, W_COLS), jnp.int32),
        pltpu.VMEM((RB, W_COLS), jnp.int32),
        pltpu.VMEM((D, RB, W_COLS), jnp.float32),
        pltpu.VMEM((D, RB, W_COLS), jnp.float32),
        pltpu.VMEM((L * 17,), jnp.float32),
        pltpu.VMEM((L * 17,), jnp.float32),
        pltpu.VMEM((L * 17,), jnp.float32),
        pltpu.SemaphoreType.DMA,
        pltpu.SemaphoreType.DMA,
        pltpu.SemaphoreType.DMA,
        pltpu.SemaphoreType.DMA,
    ],
)
def _emb_lookup(
    xt_hbm, t0_hbm, t1_hbm, t2_hbm, out_hbm, x_v0, x_v1, o_v0, o_v1,
    tab0, tab1, tab2, sin0, sin1, sout0, sout1,
):
    wid = lax.axis_index("s") * NC + lax.axis_index("c")
    pltpu.sync_copy(t0_hbm, tab0)
    pltpu.sync_copy(t1_hbm, tab1)
    pltpu.sync_copy(t2_hbm, tab2)
    tabs = (tab0, tab1, tab2)
    c0 = wid * W_COLS
    skew = lax.iota(jnp.int32, L) * 17
    x_bufs = (x_v0, x_v1)
    o_bufs = (o_v0, o_v1)
    sins = (sin0, sin1)
    souts = (sout0, sout1)

    def x_slice(jb):
        return xt_hbm.at[pl.ds(jb * RB, RB), pl.ds(c0, W_COLS)]

    def o_slice(jb):
        return out_hbm.at[pl.ds(0, D), pl.ds(jb * RB, RB), pl.ds(c0, W_COLS)]

    def compute(x_v, o_v):
        def k_body(k, carry):
            k16 = k * L
            for r in range(RB):
                xs = x_v[r, pl.ds(k16, L)] + skew
                for d in range(D):
                    t = plsc.load_gather(tabs[d], [xs])
                    o_v[d, r, pl.ds(k16, L)] = t
            return carry

        lax.fori_loop(0, W_COLS // L, k_body, 0)

    pltpu.async_copy(x_slice(0), x_bufs[0], sins[0])
    pltpu.async_copy(x_slice(1), x_bufs[1], sins[1])
    for jb in range(NCH):
        b = jb & 1
        pltpu.make_async_copy(x_slice(jb), x_bufs[b], sins[b]).wait()
        if jb >= 2:
            pltpu.make_async_copy(o_bufs[b], o_slice(jb - 2), souts[b]).wait()
        compute(x_bufs[b], o_bufs[b])
        pltpu.async_copy(o_bufs[b], o_slice(jb), souts[b])
        if jb + 2 < NCH:
            pltpu.async_copy(x_slice(jb + 2), x_bufs[b], sins[b])
    pltpu.make_async_copy(o_bufs[1], o_slice(NCH - 2), souts[1]).wait()
    pltpu.make_async_copy(o_bufs[0], o_slice(NCH - 1), souts[0]).wait()


def kernel(x, table):
    rep = jnp.broadcast_to(table.T[:, None, :], (D, L, 10))
    tt = jnp.pad(rep, ((0, 0), (0, 0), (0, 7))).reshape(D, L * 17)
    out_t = _emb_lookup(x.T.astype(jnp.int32), tt[0], tt[1], tt[2])
    return jnp.transpose(out_t, (2, 1, 0))


# speedup vs baseline: 2.2923x; 2.2923x over previous
import functools

import jax
import jax.numpy as jnp
from jax import lax
from jax.experimental import pallas as pl
from jax.experimental.pallas import tpu as pltpu
from jax.experimental.pallas import tpu_sc as plsc

NC = 2
NS = 16
L = 16
NW = NC * NS

ROWS, COLS, D = 16384, 200, 3
W_COLS = ROWS // NW
RB = 8
NCH = COLS // RB
GRP = RB * W_COLS // L

_mesh = plsc.VectorSubcoreMesh(
    core_axis_name="c", subcore_axis_name="s", num_cores=NC, num_subcores=NS
)


@functools.partial(
    pl.kernel,
    out_type=jax.ShapeDtypeStruct((D, COLS, ROWS), jnp.float32),
    mesh=_mesh,
    compiler_params=pltpu.CompilerParams(
        needs_layout_passes=False, use_tc_tiling_on_sc=True
    ),
    scratch_types=[
        pltpu.VMEM((RB, W_COLS), jnp.int32),
        pltpu.VMEM((RB, W_COLS), jnp.int32),
        pltpu.VMEM((D, RB, W_COLS), jnp.float32),
        pltpu.VMEM((D, RB, W_COLS), jnp.float32),
        pltpu.VMEM((L,), jnp.float32),
        pltpu.VMEM((L,), jnp.float32),
        pltpu.VMEM((L,), jnp.float32),
        pltpu.SemaphoreType.DMA,
        pltpu.SemaphoreType.DMA,
        pltpu.SemaphoreType.DMA,
        pltpu.SemaphoreType.DMA,
    ],
)
def _emb_lookup(
    xt_hbm, t0_hbm, t1_hbm, t2_hbm, out_hbm, x_v0, x_v1, o_v0, o_v1,
    tab0, tab1, tab2, sin0, sin1, sout0, sout1,
):
    wid = lax.axis_index("s") * NC + lax.axis_index("c")
    pltpu.sync_copy(t0_hbm, tab0)
    pltpu.sync_copy(t1_hbm, tab1)
    pltpu.sync_copy(t2_hbm, tab2)
    tabs = (tab0, tab1, tab2)
    c0 = wid * W_COLS
    x_bufs = (x_v0, x_v1)
    o_bufs = (o_v0, o_v1)
    sins = (sin0, sin1)
    souts = (sout0, sout1)

    def x_slice(jb):
        return xt_hbm.at[pl.ds(jb * RB, RB), pl.ds(c0, W_COLS)]

    def o_slice(jb):
        return out_hbm.at[pl.ds(0, D), pl.ds(jb * RB, RB), pl.ds(c0, W_COLS)]

    tvs = tuple(tabs[d][...] for d in range(D))

    def compute(x_v, o_v):
        def k_body(k, carry):
            k16 = k * L
            for r in range(RB):
                xv = x_v[r, pl.ds(k16, L)]
                for d in range(D):
                    t = jnp.take_along_axis(
                        tvs[d], xv, axis=0, mode="promise_in_bounds"
                    )
                    o_v[d, r, pl.ds(k16, L)] = t
            return carry

        lax.fori_loop(0, W_COLS // L, k_body, 0)

    pltpu.async_copy(x_slice(0), x_bufs[0], sins[0])
    pltpu.async_copy(x_slice(1), x_bufs[1], sins[1])
    for jb in range(NCH):
        b = jb & 1
        pltpu.make_async_copy(x_slice(jb), x_bufs[b], sins[b]).wait()
        if jb >= 2:
            pltpu.make_async_copy(o_bufs[b], o_slice(jb - 2), souts[b]).wait()
        compute(x_bufs[b], o_bufs[b])
        pltpu.async_copy(o_bufs[b], o_slice(jb), souts[b])
        if jb + 2 < NCH:
            pltpu.async_copy(x_slice(jb + 2), x_bufs[b], sins[b])
    pltpu.make_async_copy(o_bufs[1], o_slice(NCH - 2), souts[1]).wait()
    pltpu.make_async_copy(o_bufs[0], o_slice(NCH - 1), souts[0]).wait()


def kernel(x, table):
    tt = jnp.zeros((D, L), jnp.float32).at[:, :10].set(table.T)
    out_t = _emb_lookup(x.T.astype(jnp.int32), tt[0], tt[1], tt[2])
    return jnp.transpose(out_t, (2, 1, 0))


# speedup vs baseline: 2.4155x; 1.0538x over previous
import functools

import jax
import jax.numpy as jnp
from jax import lax
from jax.experimental import pallas as pl
from jax.experimental.pallas import tpu as pltpu
from jax.experimental.pallas import tpu_sc as plsc

NC = 2
NS = 16
L = 16
NW = NC * NS

ROWS, COLS, D = 16384, 200, 3
W_COLS = ROWS // NW
RB = 8
NCH = COLS // RB
GRP = RB * W_COLS // L

_mesh = plsc.VectorSubcoreMesh(
    core_axis_name="c", subcore_axis_name="s", num_cores=NC, num_subcores=NS
)


@functools.partial(
    pl.kernel,
    out_type=jax.ShapeDtypeStruct((D, COLS, ROWS), jnp.float32),
    mesh=_mesh,
    compiler_params=pltpu.CompilerParams(
        needs_layout_passes=False, use_tc_tiling_on_sc=True
    ),
    scratch_types=[
        pltpu.VMEM((RB, W_COLS), jnp.int32),
        pltpu.VMEM((RB, W_COLS), jnp.int32),
        pltpu.VMEM((RB, W_COLS), jnp.int32),
        pltpu.VMEM((RB, W_COLS), jnp.int32),
        pltpu.VMEM((D, RB, W_COLS), jnp.float32),
        pltpu.VMEM((D, RB, W_COLS), jnp.float32),
        pltpu.VMEM((D, RB, W_COLS), jnp.float32),
        pltpu.VMEM((D, RB, W_COLS), jnp.float32),
        pltpu.VMEM((L,), jnp.float32),
        pltpu.VMEM((L,), jnp.float32),
        pltpu.VMEM((L,), jnp.float32),
        pltpu.SemaphoreType.DMA,
        pltpu.SemaphoreType.DMA,
        pltpu.SemaphoreType.DMA,
        pltpu.SemaphoreType.DMA,
        pltpu.SemaphoreType.DMA,
        pltpu.SemaphoreType.DMA,
        pltpu.SemaphoreType.DMA,
        pltpu.SemaphoreType.DMA,
    ],
)
def _emb_lookup(
    xt_hbm, t0_hbm, t1_hbm, t2_hbm, out_hbm,
    x_v0, x_v1, x_v2, x_v3, o_v0, o_v1, o_v2, o_v3,
    tab0, tab1, tab2,
    sin0, sin1, sin2, sin3, sout0, sout1, sout2, sout3,
):
    wid = lax.axis_index("s") * NC + lax.axis_index("c")
    pltpu.sync_copy(t0_hbm, tab0)
    pltpu.sync_copy(t1_hbm, tab1)
    pltpu.sync_copy(t2_hbm, tab2)
    tabs = (tab0, tab1, tab2)
    c0 = wid * W_COLS
    x_bufs = (x_v0, x_v1, x_v2, x_v3)
    o_bufs = (o_v0, o_v1, o_v2, o_v3)
    sins = (sin0, sin1, sin2, sin3)
    souts = (sout0, sout1, sout2, sout3)
    NBUF = 4

    def x_slice(jb):
        return xt_hbm.at[pl.ds(jb * RB, RB), pl.ds(c0, W_COLS)]

    def o_slice(jb):
        return out_hbm.at[pl.ds(0, D), pl.ds(jb * RB, RB), pl.ds(c0, W_COLS)]

    tvs = tuple(tabs[d][...] for d in range(D))

    def compute(x_v, o_v):
        def k_body(k, carry):
            k16 = k * L
            for r in range(RB):
                xv = x_v[r, pl.ds(k16, L)]
                for d in range(D):
                    t = jnp.take_along_axis(
                        tvs[d], xv, axis=0, mode="promise_in_bounds"
                    )
                    o_v[d, r, pl.ds(k16, L)] = t
            return carry

        lax.fori_loop(0, W_COLS // L, k_body, 0)

    for jb in range(NBUF):
        pltpu.async_copy(x_slice(jb), x_bufs[jb], sins[jb])
    for jb in range(NCH):
        b = jb % NBUF
        pltpu.make_async_copy(x_slice(jb), x_bufs[b], sins[b]).wait()
        if jb >= NBUF:
            pltpu.make_async_copy(
                o_bufs[b], o_slice(jb - NBUF), souts[b]
            ).wait()
        compute(x_bufs[b], o_bufs[b])
        pltpu.async_copy(o_bufs[b], o_slice(jb), souts[b])
        if jb + NBUF < NCH:
            pltpu.async_copy(x_slice(jb + NBUF), x_bufs[b], sins[b])
    for jb in range(NCH - NBUF, NCH):
        b = jb % NBUF
        pltpu.make_async_copy(o_bufs[b], o_slice(jb), souts[b]).wait()


def kernel(x, table):
    tt = jnp.zeros((D, L), jnp.float32).at[:, :10].set(table.T)
    out_t = _emb_lookup(x.T.astype(jnp.int32), tt[0], tt[1], tt[2])
    return jnp.transpose(out_t, (2, 1, 0))
